# Initial kernel scaffold; baseline (speedup 1.0000x reference)
import jax
import jax.numpy as jnp
from jax.experimental import pallas as pl


def kernel(node_embeddings, edge_index_0, edge_index_1, edge_index_2, edge_index_3, W_msg, b_msg, W_upd, b_upd):
    raise NotImplementedError("write your pallas kernel here")


# speedup vs baseline: 5.5469x; 5.5469x over previous
import functools

import jax
import jax.numpy as jnp
from jax import lax
from jax.experimental import pallas as pl
from jax.experimental.pallas import tpu as pltpu
from jax.experimental.pallas import tpu_sc as plsc

_N, _D, _E, _R = 10000, 128, 80000, 4
_NC, _NS = 2, 16
_NW = _NC * _NS
_CH = 128
_NCHUNK = _E // _CH
_CPT = -(-_NCHUNK // _NW)
_BN = 400

_ROWS_A, _ROWS_B = 640, _N - 15 * 640


def _precompute(x, W_msg, b_msg):
    def body(x_ref, w_ref, bm_ref, yt_ref, yb_ref):
        xb = x_ref[...]
        for r in range(_R):
            w = w_ref[r]
            yt_ref[r] = (
                jnp.dot(xb, w[:_D], preferred_element_type=jnp.float32)
                + bm_ref[r][None, :]
            )
            yb_ref[r] = jnp.dot(xb, w[_D:], preferred_element_type=jnp.float32)

    return pl.pallas_call(
        body,
        grid=(_N // _BN,),
        in_specs=[
            pl.BlockSpec((_BN, _D), lambda i: (i, 0)),
            pl.BlockSpec((_R, 2 * _D, _D), lambda i: (0, 0, 0)),
            pl.BlockSpec((_R, _D), lambda i: (0, 0)),
        ],
        out_specs=[
            pl.BlockSpec((_R, _BN, _D), lambda i: (0, i, 0)),
            pl.BlockSpec((_R, _BN, _D), lambda i: (0, i, 0)),
        ],
        out_shape=[jax.ShapeDtypeStruct((_R, _N, _D), jnp.float32)] * 2,
    )(x, W_msg, b_msg)


@functools.partial(
    pl.kernel,
    out_type=jax.ShapeDtypeStruct((_NC * _N, _D), jnp.float32),
    mesh=plsc.VectorSubcoreMesh(core_axis_name="c", subcore_axis_name="s"),
    scratch_types=[
        pltpu.VMEM((_CH,), jnp.int32),
        pltpu.VMEM((_CH,), jnp.int32),
        pltpu.VMEM((_CH, _D), jnp.float32),
        pltpu.VMEM((_CH, _D), jnp.float32),
        pltpu.VMEM_SHARED((_N, _D), jnp.float32),
        pltpu.SemaphoreType.DMA,
        pltpu.SemaphoreType.DMA,
    ],
)
def _sc_aggregate(yt, yb, src, dst, zrows, out,
                  sidx, didx, arows, brows, agg, sem_a, sem_b):
    c = lax.axis_index("c")
    s = lax.axis_index("s")
    wid = s * _NC + c
    row0 = s * _ROWS_A

    @pl.when(s < 15)
    def _():
        pltpu.sync_copy(zrows, agg.at[pl.ds(row0, _ROWS_A)])

    @pl.when(s == 15)
    def _():
        pltpu.sync_copy(zrows.at[pl.ds(0, _ROWS_B)], agg.at[pl.ds(row0, _ROWS_B)])

    plsc.subcore_barrier()

    for r in range(_R):
        def chunk_body(k, _, r=r):
            g = k * _NW + wid

            @pl.when(g < _NCHUNK)
            def _():
                base = g * _CH
                pltpu.sync_copy(src.at[r, pl.ds(base, _CH)], sidx)
                pltpu.sync_copy(dst.at[r, pl.ds(base, _CH)], didx)
                ca = pltpu.async_copy(yt.at[r].at[sidx], arows, sem_a)
                cb = pltpu.async_copy(yb.at[r].at[didx], brows, sem_b)
                ca.wait()
                cb.wait()

                def row_body(i, _):
                    for j in range(_D // 16):
                        a = arows[i, pl.ds(j * 16, 16)]
                        b = brows[i, pl.ds(j * 16, 16)]
                        arows[i, pl.ds(j * 16, 16)] = jnp.maximum(a + b, 0.0)
                    return 0

                lax.fori_loop(0, _CH, row_body, 0)
                pltpu.sync_copy(arows, agg.at[didx], add=True)

            return 0

        lax.fori_loop(0, _CPT, chunk_body, 0)

    plsc.subcore_barrier()

    @pl.when(s < 15)
    def _():
        pltpu.sync_copy(agg.at[pl.ds(row0, _ROWS_A)],
                        out.at[pl.ds(c * _N + row0, _ROWS_A)])

    @pl.when(s == 15)
    def _():
        pltpu.sync_copy(agg.at[pl.ds(row0, _ROWS_B)],
                        out.at[pl.ds(c * _N + row0, _ROWS_B)])


def _update(x, aggp, W_upd, b_upd):
    def body(x_ref, a_ref, w_ref, b_ref, o_ref):
        xb = x_ref[...]
        ab = a_ref[0] + a_ref[1]
        acc = jnp.dot(xb, w_ref[: _D], preferred_element_type=jnp.float32)
        acc = acc + jnp.dot(ab, w_ref[_D:], preferred_element_type=jnp.float32)
        o_ref[...] = jnp.maximum(acc + b_ref[0][None, :], 0.0)

    return pl.pallas_call(
        body,
        grid=(_N // _BN,),
        in_specs=[
            pl.BlockSpec((_BN, _D), lambda i: (i, 0)),
            pl.BlockSpec((_NC, _BN, _D), lambda i: (0, i, 0)),
            pl.BlockSpec((2 * _D, _D), lambda i: (0, 0)),
            pl.BlockSpec((1, _D), lambda i: (0, 0)),
        ],
        out_specs=pl.BlockSpec((_BN, _D), lambda i: (i, 0)),
        out_shape=jax.ShapeDtypeStruct((_N, _D), jnp.float32),
    )(x, aggp, W_upd, b_upd.reshape(1, _D))


def kernel(node_embeddings, edge_index_0, edge_index_1, edge_index_2,
           edge_index_3, W_msg, b_msg, W_upd, b_upd):
    x = node_embeddings
    yt, yb = _precompute(x, W_msg, b_msg)
    ei = jnp.stack([edge_index_0, edge_index_1, edge_index_2, edge_index_3])
    src = ei[:, 0, :]
    dst = ei[:, 1, :]
    zrows = jnp.zeros((_ROWS_A, _D), jnp.float32)
    aggp = _sc_aggregate(yt, yb, src, dst, zrows)
    aggp = aggp.reshape(_NC, _N, _D)
    return _update(x, aggp, W_upd, b_upd)


# speedup vs baseline: 7.8124x; 1.4084x over previous
import functools

import jax
import jax.numpy as jnp
from jax import lax
from jax.experimental import pallas as pl
from jax.experimental.pallas import tpu as pltpu
from jax.experimental.pallas import tpu_sc as plsc

_N, _D, _E, _R = 10000, 128, 80000, 4
_NC, _NS = 2, 16
_NW = _NC * _NS
_CH = 64
_NCHUNK = _E // _CH
_CPT = -(-_NCHUNK // _NW)
_BN = 400

_ROWS_A, _ROWS_B = 640, _N - 15 * 640


def _precompute(x, W_msg, b_msg):
    def body(x_ref, w_ref, bm_ref, yt_ref, yb_ref):
        xb = x_ref[...]
        for r in range(_R):
            w = w_ref[r]
            yt_ref[r] = (
                jnp.dot(xb, w[:_D], preferred_element_type=jnp.float32)
                + bm_ref[r][None, :]
            )
            yb_ref[r] = jnp.dot(xb, w[_D:], preferred_element_type=jnp.float32)

    return pl.pallas_call(
        body,
        grid=(_N // _BN,),
        in_specs=[
            pl.BlockSpec((_BN, _D), lambda i: (i, 0)),
            pl.BlockSpec((_R, 2 * _D, _D), lambda i: (0, 0, 0)),
            pl.BlockSpec((_R, _D), lambda i: (0, 0)),
        ],
        out_specs=[
            pl.BlockSpec((_R, _BN, _D), lambda i: (0, i, 0)),
            pl.BlockSpec((_R, _BN, _D), lambda i: (0, i, 0)),
        ],
        out_shape=[jax.ShapeDtypeStruct((_R, _N, _D), jnp.float32)] * 2,
    )(x, W_msg, b_msg)


_TOT = _R * _NCHUNK
_CPW = _TOT // _NW
_XTR = _TOT - _CPW * _NW


@functools.partial(
    pl.kernel,
    out_type=jax.ShapeDtypeStruct((_NC * _N, _D), jnp.float32),
    mesh=plsc.VectorSubcoreMesh(core_axis_name="c", subcore_axis_name="s"),
    scratch_types=[
        pltpu.VMEM((_CH,), jnp.int32),
        pltpu.VMEM((_CH,), jnp.int32),
        pltpu.VMEM((_CH,), jnp.int32),
        pltpu.VMEM((_CH,), jnp.int32),
        pltpu.VMEM((_CH,), jnp.int32),
        pltpu.VMEM((_CH,), jnp.int32),
        pltpu.VMEM((_CH, _D), jnp.float32),
        pltpu.VMEM((_CH, _D), jnp.float32),
        pltpu.VMEM((_CH, _D), jnp.float32),
        pltpu.VMEM((_CH, _D), jnp.float32),
        pltpu.VMEM((_CH, _D), jnp.float32),
        pltpu.VMEM((_CH, _D), jnp.float32),
        pltpu.VMEM_SHARED((_N, _D), jnp.float32),
        pltpu.SemaphoreType.DMA,
        pltpu.SemaphoreType.DMA,
        pltpu.SemaphoreType.DMA,
        pltpu.SemaphoreType.DMA,
    ],
)
def _sc_aggregate(yt, yb, src, dst, zrows, out,
                  sidx0, sidx1, didx0, didx1, didx2, didx3,
                  ar0, ar1, br0, br1, mb0, mb1, agg,
                  sg0, sg1, ss0, ss1):
    sidx = [sidx0, sidx1]
    didx = [didx0, didx1, didx2, didx3]
    ar = [ar0, ar1]
    br = [br0, br1]
    mb = [mb0, mb1]
    sg = [sg0, sg1]
    ss = [ss0, ss1]

    c = lax.axis_index("c")
    s = lax.axis_index("s")
    wid = s * _NC + c
    row0 = s * _ROWS_A

    @pl.when(s < 15)
    def _():
        pltpu.sync_copy(zrows, agg.at[pl.ds(row0, _ROWS_A)])

    @pl.when(s == 15)
    def _():
        pltpu.sync_copy(zrows.at[pl.ds(0, _ROWS_B)], agg.at[pl.ds(row0, _ROWS_B)])

    plsc.subcore_barrier()

    start = jnp.where(wid < _XTR, wid * (_CPW + 1), wid * _CPW + _XTR)
    cnt = jnp.where(wid < _XTR, _CPW + 1, _CPW)

    def issue(k, p, dd):
        @pl.when(k < cnt)
        def _():
            cid = start + k
            r = cid // _NCHUNK
            base = (cid % _NCHUNK) * _CH
            pltpu.sync_copy(src.at[r, pl.ds(base, _CH)], sidx[p])
            pltpu.sync_copy(dst.at[r, pl.ds(base, _CH)], didx[dd])
            pltpu.async_copy(yt.at[r].at[sidx[p]], ar[p], sg[p])
            pltpu.async_copy(yb.at[r].at[didx[dd]], br[p], sg[p])

    def wait_gather(k, p, dd):
        @pl.when(k < cnt)
        def _():
            pltpu.make_async_copy(yt.at[0].at[sidx[p]], ar[p], sg[p]).wait()
            pltpu.make_async_copy(yb.at[0].at[didx[dd]], br[p], sg[p]).wait()

    def wait_scatter(k, p, dd):
        @pl.when(jnp.logical_and(k >= 0, k < cnt))
        def _():
            pltpu.make_async_copy(mb[p], agg.at[didx[dd]], ss[p]).wait()

    def compute(k, p):
        @pl.when(k < cnt)
        def _():
            a_r, b_r, m_r = ar[p], br[p], mb[p]

            def row2(i, _):
                i0 = i * 2
                for oo in range(2):
                    for j in range(_D // 16):
                        sl = pl.ds(j * 16, 16)
                        m_r[i0 + oo, sl] = jnp.maximum(
                            a_r[i0 + oo, sl] + b_r[i0 + oo, sl], 0.0)
                return 0

            lax.fori_loop(0, _CH // 2, row2, 0)

    def scatter(k, p, dd):
        @pl.when(k < cnt)
        def _():
            pltpu.async_copy(mb[p], agg.at[didx[dd]], ss[p], add=True)

    issue(0, 0, 0)
    issue(1, 1, 1)

    def quad(j, _):
        for o in range(4):
            k = 4 * j + o
            p = o % 2
            wait_gather(k, p, o)
            wait_scatter(k - 2, p, (o + 2) % 4)
            compute(k, p)
            scatter(k, p, o)
            issue(k + 2, p, (o + 2) % 4)
        return 0

    lax.fori_loop(0, -(-(_CPW + 3) // 4), quad, 0)

    plsc.subcore_barrier()

    @pl.when(s < 15)
    def _():
        pltpu.sync_copy(agg.at[pl.ds(row0, _ROWS_A)],
                        out.at[pl.ds(c * _N + row0, _ROWS_A)])

    @pl.when(s == 15)
    def _():
        pltpu.sync_copy(agg.at[pl.ds(row0, _ROWS_B)],
                        out.at[pl.ds(c * _N + row0, _ROWS_B)])


def _update(x, aggp, W_upd, b_upd):
    def body(x_ref, a_ref, w_ref, b_ref, o_ref):
        xb = x_ref[...]
        ab = a_ref[0] + a_ref[1]
        acc = jnp.dot(xb, w_ref[: _D], preferred_element_type=jnp.float32)
        acc = acc + jnp.dot(ab, w_ref[_D:], preferred_element_type=jnp.float32)
        o_ref[...] = jnp.maximum(acc + b_ref[0][None, :], 0.0)

    return pl.pallas_call(
        body,
        grid=(_N // _BN,),
        in_specs=[
            pl.BlockSpec((_BN, _D), lambda i: (i, 0)),
            pl.BlockSpec((_NC, _BN, _D), lambda i: (0, i, 0)),
            pl.BlockSpec((2 * _D, _D), lambda i: (0, 0)),
            pl.BlockSpec((1, _D), lambda i: (0, 0)),
        ],
        out_specs=pl.BlockSpec((_BN, _D), lambda i: (i, 0)),
        out_shape=jax.ShapeDtypeStruct((_N, _D), jnp.float32),
    )(x, aggp, W_upd, b_upd.reshape(1, _D))


def kernel(node_embeddings, edge_index_0, edge_index_1, edge_index_2,
           edge_index_3, W_msg, b_msg, W_upd, b_upd):
    x = node_embeddings
    yt, yb = _precompute(x, W_msg, b_msg)
    ei = jnp.stack([edge_index_0, edge_index_1, edge_index_2, edge_index_3])
    src = ei[:, 0, :]
    dst = ei[:, 1, :]
    zrows = jnp.zeros((_ROWS_A, _D), jnp.float32)
    aggp = _sc_aggregate(yt, yb, src, dst, zrows)
    aggp = aggp.reshape(_NC, _N, _D)
    return _update(x, aggp, W_upd, b_upd)


# speedup vs baseline: 9.5720x; 1.2252x over previous
import functools

import jax
import jax.numpy as jnp
from jax import lax
from jax.experimental import pallas as pl
from jax.experimental.pallas import tpu as pltpu
from jax.experimental.pallas import tpu_sc as plsc

_N, _D, _E, _R = 10000, 128, 80000, 4
_NC, _NS = 2, 16
_NW = _NC * _NS
_CH = 64
_NCHUNK = _E // _CH
_CPT = -(-_NCHUNK // _NW)
_BN = 400

_ROWS_A, _ROWS_B = 640, _N - 15 * 640


def _precompute(x, W_msg, b_msg):
    def body(x_ref, w_ref, bm_ref, yt_ref, yb_ref):
        xb = x_ref[...]
        for r in range(_R):
            w = w_ref[r]
            yt_ref[r] = (
                jnp.dot(xb, w[:_D], preferred_element_type=jnp.float32)
                + bm_ref[r][None, :]
            )
            yb_ref[r] = jnp.dot(xb, w[_D:], preferred_element_type=jnp.float32)

    return pl.pallas_call(
        body,
        grid=(_N // _BN,),
        in_specs=[
            pl.BlockSpec((_BN, _D), lambda i: (i, 0)),
            pl.BlockSpec((_R, 2 * _D, _D), lambda i: (0, 0, 0)),
            pl.BlockSpec((_R, _D), lambda i: (0, 0)),
        ],
        out_specs=[
            pl.BlockSpec((_R, _BN, _D), lambda i: (0, i, 0)),
            pl.BlockSpec((_R, _BN, _D), lambda i: (0, i, 0)),
        ],
        out_shape=[jax.ShapeDtypeStruct((_R, _N, _D), jnp.float32)] * 2,
    )(x, W_msg, b_msg)


_TOT = _R * _NCHUNK
_CPW = _TOT // _NW
_XTR = _TOT - _CPW * _NW


@functools.partial(
    pl.kernel,
    out_type=jax.ShapeDtypeStruct((_NC * _N, _D), jnp.float32),
    mesh=plsc.VectorSubcoreMesh(core_axis_name="c", subcore_axis_name="s"),
    scratch_types=(
        [pltpu.VMEM((_CH,), jnp.int32)] * 4 +
        [pltpu.VMEM((_CH,), jnp.int32)] * 8 +
        [pltpu.VMEM((_CH, _D), jnp.float32)] * 6 +
        [
            pltpu.VMEM_SHARED((_N, _D), jnp.float32),
            pltpu.SemaphoreType.DMA,
            pltpu.SemaphoreType.DMA,
            pltpu.SemaphoreType.DMA,
            pltpu.SemaphoreType.DMA,
            pltpu.SemaphoreType.DMA,
            pltpu.SemaphoreType.DMA,
        ]
    ),
)
def _sc_aggregate(yt, yb, srch, dsth, zrows, out,
                  u0, u1, u2, u3, x0, x1, x2, x3, x4, x5, x6, x7,
                  ar0, ar1, br0, br1, mb0, mb1, agg,
                  si0, si1, sg0, sg1, ss0, ss1):
    sidx = [u0, u1, u2, u3]
    didx = [x0, x1, x2, x3, x4, x5, x6, x7]
    ar = [ar0, ar1]
    br = [br0, br1]
    mb = [mb0, mb1]
    si = [si0, si1]
    sg = [sg0, sg1]
    ss = [ss0, ss1]

    c = lax.axis_index("c")
    s = lax.axis_index("s")
    wid = s * _NC + c
    row0 = s * _ROWS_A

    @pl.when(s < 15)
    def _():
        pltpu.sync_copy(zrows, agg.at[pl.ds(row0, _ROWS_A)])

    @pl.when(s == 15)
    def _():
        pltpu.sync_copy(zrows.at[pl.ds(0, _ROWS_B)], agg.at[pl.ds(row0, _ROWS_B)])

    plsc.subcore_barrier()

    start = jnp.where(wid < _XTR, wid * (_CPW + 1), wid * _CPW + _XTR)
    cnt = jnp.where(wid < _XTR, _CPW + 1, _CPW)

    def issue_idx(k, d4, d8, p, pred=True):
        def go():
            cid = start + k
            pltpu.async_copy(srch.at[cid], sidx[d4], si[p])
            pltpu.async_copy(dsth.at[cid], didx[d8], si[p])
        if pred:
            pl.when(k < cnt)(go)
        else:
            go()

    def issue_gather(k, d4, d8, p, pred=True):
        def go():
            pltpu.make_async_copy(srch.at[0], sidx[d4], si[p]).wait()
            pltpu.make_async_copy(dsth.at[0], didx[d8], si[p]).wait()
            r = (start + k) // _NCHUNK
            pltpu.async_copy(yt.at[r].at[sidx[d4]], ar[p], sg[p])
            pltpu.async_copy(yb.at[r].at[didx[d8]], br[p], sg[p])
        if pred:
            pl.when(k < cnt)(go)
        else:
            go()

    def wait_gather(k, d4, d8, p):
        @pl.when(k < cnt)
        def _():
            pltpu.make_async_copy(yt.at[0].at[sidx[d4]], ar[p], sg[p]).wait()
            pltpu.make_async_copy(yb.at[0].at[didx[d8]], br[p], sg[p]).wait()

    def wait_scatter(k, d8, p):
        @pl.when(jnp.logical_and(k >= 0, k < cnt))
        def _():
            pltpu.make_async_copy(mb[p], agg.at[didx[d8]], ss[p]).wait()

    def compute(k, p):
        @pl.when(k < cnt)
        def _():
            a_r, b_r, m_r = ar[p], br[p], mb[p]

            def row2(i, _):
                i0 = i * 2
                for oo in range(2):
                    for j in range(_D // 16):
                        sl = pl.ds(j * 16, 16)
                        m_r[i0 + oo, sl] = jnp.maximum(
                            a_r[i0 + oo, sl] + b_r[i0 + oo, sl], 0.0)
                return 0

            lax.fori_loop(0, _CH // 2, row2, 0)

    def scatter(k, d8, p):
        @pl.when(k < cnt)
        def _():
            pltpu.async_copy(mb[p], agg.at[didx[d8]], ss[p], add=True)

    issue_idx(0, 0, 0, 0, pred=False)
    issue_idx(1, 1, 1, 1, pred=False)
    issue_gather(0, 0, 0, 0, pred=False)
    issue_idx(2, 2, 2, 0, pred=False)
    issue_gather(1, 1, 1, 1, pred=False)
    issue_idx(3, 3, 3, 1, pred=False)

    def oct(j, _):
        for o in range(8):
            k = 8 * j + o
            p = o % 2
            wait_gather(k, o % 4, o, p)
            wait_scatter(k - 2, (o + 6) % 8, p)
            compute(k, p)
            scatter(k, o, p)
            issue_gather(k + 2, (o + 2) % 4, (o + 2) % 8, p)
            issue_idx(k + 4, o % 4, (o + 4) % 8, p)
        return 0

    lax.fori_loop(0, -(-(_CPW + 3) // 8), oct, 0)

    plsc.subcore_barrier()

    @pl.when(s < 15)
    def _():
        pltpu.sync_copy(agg.at[pl.ds(row0, _ROWS_A)],
                        out.at[pl.ds(c * _N + row0, _ROWS_A)])

    @pl.when(s == 15)
    def _():
        pltpu.sync_copy(agg.at[pl.ds(row0, _ROWS_B)],
                        out.at[pl.ds(c * _N + row0, _ROWS_B)])


def _update(x, aggp, W_upd, b_upd):
    def body(x_ref, a_ref, w_ref, b_ref, o_ref):
        xb = x_ref[...]
        ab = a_ref[0] + a_ref[1]
        acc = jnp.dot(xb, w_ref[: _D], preferred_element_type=jnp.float32)
        acc = acc + jnp.dot(ab, w_ref[_D:], preferred_element_type=jnp.float32)
        o_ref[...] = jnp.maximum(acc + b_ref[0][None, :], 0.0)

    return pl.pallas_call(
        body,
        grid=(_N // _BN,),
        in_specs=[
            pl.BlockSpec((_BN, _D), lambda i: (i, 0)),
            pl.BlockSpec((_NC, _BN, _D), lambda i: (0, i, 0)),
            pl.BlockSpec((2 * _D, _D), lambda i: (0, 0)),
            pl.BlockSpec((1, _D), lambda i: (0, 0)),
        ],
        out_specs=pl.BlockSpec((_BN, _D), lambda i: (i, 0)),
        out_shape=jax.ShapeDtypeStruct((_N, _D), jnp.float32),
    )(x, aggp, W_upd, b_upd.reshape(1, _D))


def kernel(node_embeddings, edge_index_0, edge_index_1, edge_index_2,
           edge_index_3, W_msg, b_msg, W_upd, b_upd):
    x = node_embeddings
    yt, yb = _precompute(x, W_msg, b_msg)
    ei = jnp.stack([edge_index_0, edge_index_1, edge_index_2, edge_index_3])
    srch = ei[:, 0, :].reshape(_R * _NCHUNK, _CH)
    dsth = ei[:, 1, :].reshape(_R * _NCHUNK, _CH)
    zrows = jnp.zeros((_ROWS_A, _D), jnp.float32)
    aggp = _sc_aggregate(yt, yb, srch, dsth, zrows)
    aggp = aggp.reshape(_NC, _N, _D)
    return _update(x, aggp, W_upd, b_upd)


# speedup vs baseline: 9.6459x; 1.0077x over previous
import functools

import jax
import jax.numpy as jnp
from jax import lax
from jax.experimental import pallas as pl
from jax.experimental.pallas import tpu as pltpu
from jax.experimental.pallas import tpu_sc as plsc

_N, _D, _E, _R = 10000, 128, 80000, 4
_NC, _NS = 2, 16
_NW = _NC * _NS
_CH = 64
_NCHUNK = _E // _CH
_BN = 400

_ROWS_A, _ROWS_B = 640, _N - 15 * 640

_TOT = _R * _NCHUNK
_CPW = _TOT // _NW
_XTR = _TOT - _CPW * _NW
_EPAD = 5120


def _precompute(x, W_msg, b_msg):
    def body(x_ref, w_ref, bm_ref, yy_ref):
        xb = x_ref[...]
        for r in range(_R):
            w = w_ref[r]
            yy_ref[r, 0] = (
                jnp.dot(xb, w[:_D], preferred_element_type=jnp.float32)
                + bm_ref[r][None, :]
            )
            yy_ref[r, 1] = jnp.dot(xb, w[_D:], preferred_element_type=jnp.float32)

    return pl.pallas_call(
        body,
        grid=(_N // _BN,),
        in_specs=[
            pl.BlockSpec((_BN, _D), lambda i: (i, 0)),
            pl.BlockSpec((_R, 2 * _D, _D), lambda i: (0, 0, 0)),
            pl.BlockSpec((_R, _D), lambda i: (0, 0)),
        ],
        out_specs=pl.BlockSpec((_R, 2, _BN, _D), lambda i: (0, 0, i, 0)),
        out_shape=jax.ShapeDtypeStruct((_R, 2, _N, _D), jnp.float32),
    )(x, W_msg, b_msg)


@functools.partial(
    pl.kernel,
    out_type=jax.ShapeDtypeStruct((_NC * _N, _D), jnp.float32),
    mesh=plsc.VectorSubcoreMesh(core_axis_name="c", subcore_axis_name="s"),
    scratch_types=(
        [pltpu.VMEM((8, 2 * _CH), jnp.int32)] * 2 +
        [pltpu.VMEM((_CH,), jnp.int32)] * 2 +
        [pltpu.VMEM((2 * _CH, _D), jnp.float32)] * 2 +
        [
            pltpu.VMEM((_CH, _D), jnp.float32),
            pltpu.VMEM_SHARED((_N, _D), jnp.float32),
            pltpu.SemaphoreType.DMA,
            pltpu.SemaphoreType.DMA,
            pltpu.SemaphoreType.DMA,
            pltpu.SemaphoreType.DMA,
            pltpu.SemaphoreType.DMA,
        ]
    ),
)
def _sc_aggregate(yy, esd, zrows, out,
                  blk0, blk1, dx0, dx1, cb0, cb1, mb, agg,
                  sb0, sb1, sg0, sg1, ss):
    blk = [blk0, blk1]
    dx = [dx0, dx1]
    cb = [cb0, cb1]
    sb = [sb0, sb1]
    sg = [sg0, sg1]

    c = lax.axis_index("c")
    s = lax.axis_index("s")
    wid = s * _NC + c
    row0 = s * _ROWS_A

    @pl.when(s < 15)
    def _():
        pltpu.sync_copy(zrows, agg.at[pl.ds(row0, _ROWS_A)])

    @pl.when(s == 15)
    def _():
        pltpu.sync_copy(zrows.at[pl.ds(0, _ROWS_B)], agg.at[pl.ds(row0, _ROWS_B)])

    plsc.subcore_barrier()

    nb = _TOT // 8
    bpw, bx = nb // _NW, nb % _NW
    bstart = jnp.where(wid < bx, wid * (bpw + 1), wid * bpw + bx)
    start = 8 * bstart
    cnt = 8 * jnp.where(wid < bx, bpw + 1, bpw)

    def issue_block(b, ring, pred=True):
        def go():
            pltpu.async_copy(esd.at[pl.ds(start + 8 * b, 8)], blk[ring], sb[ring])
        if pred:
            pl.when(8 * b < cnt)(go)
        else:
            go()

    def wait_block(b, ring):
        @pl.when(8 * b < cnt)
        def _():
            pltpu.make_async_copy(esd.at[pl.ds(0, 8)], blk[ring], sb[ring]).wait()

    def issue_gather(k, ring, row, p, pred=True):
        def go():
            r = (start + k) // _NCHUNK
            pltpu.async_copy(yy.at[r].at[blk[ring].at[row]], cb[p], sg[p])
        if pred:
            pl.when(k < cnt)(go)
        else:
            go()

    def wait_gather(k, ring, row, p):
        @pl.when(k < cnt)
        def _():
            pltpu.make_async_copy(yy.at[0].at[blk[ring].at[row]], cb[p], sg[p]).wait()

    def wait_scatter(k, p2):
        @pl.when(jnp.logical_and(k >= 0, k < cnt))
        def _():
            pltpu.make_async_copy(mb, agg.at[dx[p2]], ss).wait()

    def compute(k, ring, row, p, p2):
        @pl.when(k < cnt)
        def _():
            for j in range(_CH // 16):
                sl = pl.ds(_CH + j * 16, 16)
                dx[p2][pl.ds(j * 16, 16)] = blk[ring][row, sl] - _N

            c_r = cb[p]

            def row2(i, _):
                i0 = i * 2
                for oo in range(2):
                    for j in range(_D // 16):
                        sl = pl.ds(j * 16, 16)
                        mb[i0 + oo, sl] = jnp.maximum(
                            c_r[i0 + oo, sl] + c_r[_CH + i0 + oo, sl], 0.0)
                return 0

            lax.fori_loop(0, _CH // 2, row2, 0)

    def scatter(k, p2):
        @pl.when(k < cnt)
        def _():
            pltpu.async_copy(mb, agg.at[dx[p2]], ss, add=True)

    issue_block(0, 0, pred=False)
    issue_block(1, 1, pred=False)
    wait_block(0, 0)
    issue_gather(0, 0, 0, 0, pred=False)
    issue_gather(1, 0, 1, 1, pred=False)

    def body16(jj, _):
        for t in range(16):
            k = 16 * jj + t
            ring, row, p, p2 = t // 8, t % 8, t % 2, t % 2
            t2 = (t + 2) % 16
            wait_gather(k, ring, row, p)
            wait_scatter(k - 1, (t + 1) % 2)
            compute(k, ring, row, p, p2)
            scatter(k, p2)
            if t == 6:
                wait_block(2 * jj + 1, 1)
            if t == 14:
                wait_block(2 * jj + 2, 0)
            issue_gather(k + 2, t2 // 8, t2 % 8, p)
            if t == 7:
                issue_block(2 * jj + 2, 0)
            if t == 15:
                issue_block(2 * jj + 3, 1)
        return 0

    lax.fori_loop(0, -(-(8 * (_TOT // 8 // _NW + 1) + 2) // 16), body16, 0)

    plsc.subcore_barrier()

    @pl.when(s < 15)
    def _():
        pltpu.sync_copy(agg.at[pl.ds(row0, _ROWS_A)],
                        out.at[pl.ds(c * _N + row0, _ROWS_A)])

    @pl.when(s == 15)
    def _():
        pltpu.sync_copy(agg.at[pl.ds(row0, _ROWS_B)],
                        out.at[pl.ds(c * _N + row0, _ROWS_B)])


def _update(x, aggp, W_upd, b_upd):
    def body(x_ref, a_ref, w_ref, b_ref, o_ref):
        xb = x_ref[...]
        ab = a_ref[0] + a_ref[1]
        acc = jnp.dot(xb, w_ref[: _D], preferred_element_type=jnp.float32)
        acc = acc + jnp.dot(ab, w_ref[_D:], preferred_element_type=jnp.float32)
        o_ref[...] = jnp.maximum(acc + b_ref[0][None, :], 0.0)

    return pl.pallas_call(
        body,
        grid=(_N // _BN,),
        in_specs=[
            pl.BlockSpec((_BN, _D), lambda i: (i, 0)),
            pl.BlockSpec((_NC, _BN, _D), lambda i: (0, i, 0)),
            pl.BlockSpec((2 * _D, _D), lambda i: (0, 0)),
            pl.BlockSpec((1, _D), lambda i: (0, 0)),
        ],
        out_specs=pl.BlockSpec((_BN, _D), lambda i: (i, 0)),
        out_shape=jax.ShapeDtypeStruct((_N, _D), jnp.float32),
    )(x, aggp, W_upd, b_upd.reshape(1, _D))


def kernel(node_embeddings, edge_index_0, edge_index_1, edge_index_2,
           edge_index_3, W_msg, b_msg, W_upd, b_upd):
    x = node_embeddings
    yy = _precompute(x, W_msg, b_msg).reshape(_R, 2 * _N, _D)
    ei = jnp.stack([edge_index_0, edge_index_1, edge_index_2, edge_index_3])
    esd = jnp.concatenate(
        [ei[:, 0, :].reshape(_R, _NCHUNK, _CH),
         ei[:, 1, :].reshape(_R, _NCHUNK, _CH) + _N], axis=-1,
    ).reshape(_TOT, 2 * _CH)
    esd = jnp.pad(esd, ((0, _EPAD - _TOT), (0, 0)))
    zrows = jnp.zeros((_ROWS_A, _D), jnp.float32)
    aggp = _sc_aggregate(yy, esd, zrows)
    aggp = aggp.reshape(_NC, _N, _D)
    return _update(x, aggp, W_upd, b_upd)
